# baseline (speedup 1.0000x reference)
import functools

import jax
import jax.numpy as jnp
from jax import lax
from jax.experimental import pallas as pl
from jax.experimental.pallas import tpu as pltpu
from jax.experimental.pallas import tpu_sc as plsc

DIM = 128
TOTAL = 4096 * 50
NUM_WORKERS = 32
PER_WORKER = TOTAL // NUM_WORKERS
CHUNK = 160
NCHUNK = PER_WORKER // CHUNK
NBUF = 5
GROUPS = NCHUNK // NBUF

_mesh = plsc.VectorSubcoreMesh(core_axis_name="c", subcore_axis_name="s")


@functools.partial(
    pl.kernel,
    out_type=jax.ShapeDtypeStruct((TOTAL, DIM), jnp.float32),
    mesh=_mesh,
    scratch_types=[
        pltpu.VMEM((PER_WORKER,), jnp.int32),
        [pltpu.VMEM((CHUNK, DIM), jnp.float32) for _ in range(NBUF)],
        [pltpu.SemaphoreType.DMA for _ in range(NBUF)],
        [pltpu.SemaphoreType.DMA for _ in range(NBUF)],
    ],
)
def _gather_kernel(idx_hbm, table_hbm, out_hbm, idx_v, bufs, gsems, wsems):
    wid = lax.axis_index("s") * 2 + lax.axis_index("c")
    base = wid * PER_WORKER
    pltpu.sync_copy(idx_hbm.at[pl.ds(base, PER_WORKER)], idx_v)

    def start_g(j, b):
        pltpu.async_copy(
            table_hbm.at[idx_v.at[pl.ds(j * CHUNK, CHUNK)]], bufs[b], gsems[b]
        )

    def wait_g(b):
        pltpu.make_async_copy(
            table_hbm.at[pl.ds(0, CHUNK)], bufs[b], gsems[b]
        ).wait()

    def start_w(j, b):
        pltpu.async_copy(
            bufs[b], out_hbm.at[pl.ds(base + j * CHUNK, CHUNK)], wsems[b]
        )

    def wait_w(b):
        pltpu.make_async_copy(
            bufs[b], out_hbm.at[pl.ds(base, CHUNK)], wsems[b]
        ).wait()

    for b in range(NBUF - 1):
        start_g(b, b)

    def outer(o, _):
        j0 = o * NBUF
        for b in range(NBUF):
            j = j0 + b
            wait_g(b)
            start_w(j, b)
            jn = j + NBUF - 1
            bn = (b - 1) % NBUF

            @pl.when(jnp.logical_and(jn < NCHUNK, j >= 1))
            def _():
                wait_w(bn)

            @pl.when(jn < NCHUNK)
            def _():
                start_g(jn, bn)
        return 0

    lax.fori_loop(0, GROUPS, outer, 0)
    for b in range(NBUF):
        wait_w(b)


def kernel(input, length, table):
    b, h = input.shape
    idx = input.astype(jnp.int32).T.reshape(TOTAL)
    out = _gather_kernel(idx, table)
    return out.reshape(h, b, DIM).transpose(1, 0, 2), length


# speedup vs baseline: 1.0155x; 1.0155x over previous
import functools

import jax
import jax.numpy as jnp
from jax import lax
from jax.experimental import pallas as pl
from jax.experimental.pallas import tpu as pltpu
from jax.experimental.pallas import tpu_sc as plsc

DIM = 128
TOTAL = 4096 * 50
NUM_WORKERS = 32
PER_WORKER = TOTAL // NUM_WORKERS
CHUNK = 80
NCHUNK = PER_WORKER // CHUNK
NBUF = 10
GROUPS = NCHUNK // NBUF

_mesh = plsc.VectorSubcoreMesh(core_axis_name="c", subcore_axis_name="s")


@functools.partial(
    pl.kernel,
    out_type=jax.ShapeDtypeStruct((TOTAL, DIM), jnp.float32),
    mesh=_mesh,
    scratch_types=[
        pltpu.VMEM((PER_WORKER,), jnp.int32),
        [pltpu.VMEM((CHUNK, DIM), jnp.float32) for _ in range(NBUF)],
        [pltpu.SemaphoreType.DMA for _ in range(NBUF)],
        [pltpu.SemaphoreType.DMA for _ in range(NBUF)],
    ],
)
def _gather_kernel(idx_hbm, table_hbm, out_hbm, idx_v, bufs, gsems, wsems):
    wid = lax.axis_index("s") * 2 + lax.axis_index("c")
    base = wid * PER_WORKER
    pltpu.sync_copy(idx_hbm.at[pl.ds(base, PER_WORKER)], idx_v)

    def start_g(j, b):
        pltpu.async_copy(
            table_hbm.at[idx_v.at[pl.ds(j * CHUNK, CHUNK)]], bufs[b], gsems[b]
        )

    def wait_g(b):
        pltpu.make_async_copy(
            table_hbm.at[pl.ds(0, CHUNK)], bufs[b], gsems[b]
        ).wait()

    def start_w(j, b):
        pltpu.async_copy(
            bufs[b], out_hbm.at[pl.ds(base + j * CHUNK, CHUNK)], wsems[b]
        )

    def wait_w(b):
        pltpu.make_async_copy(
            bufs[b], out_hbm.at[pl.ds(base, CHUNK)], wsems[b]
        ).wait()

    for b in range(NBUF - 1):
        start_g(b, b)

    def outer(o, _):
        j0 = o * NBUF
        for b in range(NBUF):
            j = j0 + b
            wait_g(b)
            start_w(j, b)
            jn = j + NBUF - 1
            bn = (b - 1) % NBUF

            @pl.when(jnp.logical_and(jn < NCHUNK, j >= 1))
            def _():
                wait_w(bn)

            @pl.when(jn < NCHUNK)
            def _():
                start_g(jn, bn)
        return 0

    lax.fori_loop(0, GROUPS, outer, 0)
    for b in range(NBUF):
        wait_w(b)


def kernel(input, length, table):
    b, h = input.shape
    idx = input.astype(jnp.int32).T.reshape(TOTAL)
    out = _gather_kernel(idx, table)
    return out.reshape(h, b, DIM).transpose(1, 0, 2), length
